# baseline (speedup 1.0000x reference)
import jax
import jax.numpy as jnp
from jax.experimental import pallas as pl
from jax.experimental.pallas import tpu as pltpu


def _support_body(x_ref, w_ref, o_ref):
    o_ref[...] = jnp.dot(x_ref[...], w_ref[...],
                         preferred_element_type=jnp.float32)


def _spmm_body(adj_ref, s_ref, o_ref):
    o_ref[...] = jnp.dot(adj_ref[...], s_ref[...],
                         preferred_element_type=jnp.float32)


def _largest_divisor(n, target, step=8):
    best = step
    d = step
    while d <= target:
        if n % d == 0:
            best = d
        d += step
    return best


@jax.jit
def kernel(input, adj, weight):
    n, d_in = input.shape
    d_out = weight.shape[1]

    bm_s = _largest_divisor(n, 2000)
    support = pl.pallas_call(
        _support_body,
        grid=(n // bm_s,),
        in_specs=[
            pl.BlockSpec((bm_s, d_in), lambda i: (i, 0)),
            pl.BlockSpec((d_in, d_out), lambda i: (0, 0)),
        ],
        out_specs=pl.BlockSpec((bm_s, d_out), lambda i: (i, 0)),
        out_shape=jax.ShapeDtypeStruct((n, d_out), jnp.float32),
        compiler_params=pltpu.CompilerParams(
            dimension_semantics=("parallel",)),
    )(input, weight)

    bm = _largest_divisor(n, 200)
    out = pl.pallas_call(
        _spmm_body,
        grid=(n // bm,),
        in_specs=[
            pl.BlockSpec((bm, n), lambda i: (i, 0)),
            pl.BlockSpec((n, d_out), lambda i: (0, 0)),
        ],
        out_specs=pl.BlockSpec((bm, d_out), lambda i: (i, 0)),
        out_shape=jax.ShapeDtypeStruct((n, d_out), jnp.float32),
        compiler_params=pltpu.CompilerParams(
            dimension_semantics=("parallel",)),
    )(adj, support)
    return out


# speedup vs baseline: 1.0183x; 1.0183x over previous
import jax
import jax.numpy as jnp
from jax.experimental import pallas as pl
from jax.experimental.pallas import tpu as pltpu


def _support_body(x_ref, w_ref, o_ref):
    o_ref[...] = jnp.dot(x_ref[...], w_ref[...],
                         preferred_element_type=jnp.float32
                         ).astype(jnp.bfloat16)


def _spmm_body(adj_ref, s_ref, o_ref):
    o_ref[...] = jnp.dot(adj_ref[...].astype(jnp.bfloat16), s_ref[...],
                         preferred_element_type=jnp.float32)


def _largest_divisor(n, target, step=8):
    best = step
    d = step
    while d <= target:
        if n % d == 0:
            best = d
        d += step
    return best


@jax.jit
def kernel(input, adj, weight):
    n, d_in = input.shape
    d_out = weight.shape[1]

    bm_s = _largest_divisor(n, 2000)
    support = pl.pallas_call(
        _support_body,
        grid=(n // bm_s,),
        in_specs=[
            pl.BlockSpec((bm_s, d_in), lambda i: (i, 0)),
            pl.BlockSpec((d_in, d_out), lambda i: (0, 0)),
        ],
        out_specs=pl.BlockSpec((bm_s, d_out), lambda i: (i, 0)),
        out_shape=jax.ShapeDtypeStruct((n, d_out), jnp.bfloat16),
        compiler_params=pltpu.CompilerParams(
            dimension_semantics=("parallel",)),
    )(input, weight)

    bm = _largest_divisor(n, 200)
    out = pl.pallas_call(
        _spmm_body,
        grid=(n // bm,),
        in_specs=[
            pl.BlockSpec((bm, n), lambda i: (i, 0)),
            pl.BlockSpec((n, d_out), lambda i: (0, 0)),
        ],
        out_specs=pl.BlockSpec((bm, d_out), lambda i: (i, 0)),
        out_shape=jax.ShapeDtypeStruct((n, d_out), jnp.float32),
        compiler_params=pltpu.CompilerParams(
            dimension_semantics=("parallel",)),
    )(adj, support)
    return out


# speedup vs baseline: 1.0576x; 1.0386x over previous
import jax
import jax.numpy as jnp
from jax.experimental import pallas as pl
from jax.experimental.pallas import tpu as pltpu


def _fused_body(adj_ref, x_ref, w_ref, o_ref, sup_ref):
    @pl.when(pl.program_id(0) == 0)
    def _():
        sup_ref[...] = jnp.dot(
            x_ref[...].astype(jnp.bfloat16),
            w_ref[...].astype(jnp.bfloat16),
            preferred_element_type=jnp.float32).astype(jnp.bfloat16)

    o_ref[...] = jnp.dot(adj_ref[...].astype(jnp.bfloat16), sup_ref[...],
                         preferred_element_type=jnp.float32)


def _largest_divisor(n, target, step=8):
    best = step
    d = step
    while d <= target:
        if n % d == 0:
            best = d
        d += step
    return best


@jax.jit
def kernel(input, adj, weight):
    n, d_in = input.shape
    d_out = weight.shape[1]

    bm = _largest_divisor(n, 200)
    out = pl.pallas_call(
        _fused_body,
        grid=(n // bm,),
        in_specs=[
            pl.BlockSpec((bm, n), lambda i: (i, 0)),
            pl.BlockSpec((n, d_in), lambda i: (0, 0)),
            pl.BlockSpec((d_in, d_out), lambda i: (0, 0)),
        ],
        out_specs=pl.BlockSpec((bm, d_out), lambda i: (i, 0)),
        out_shape=jax.ShapeDtypeStruct((n, d_out), jnp.float32),
        scratch_shapes=[pltpu.VMEM((n, d_out), jnp.bfloat16)],
        compiler_params=pltpu.CompilerParams(
            dimension_semantics=("arbitrary",)),
    )(adj, input, weight)
    return out
